# baseline (speedup 1.0000x reference)
import functools

import jax
import jax.numpy as jnp
from jax import lax
from jax.experimental import pallas as pl
from jax.experimental.pallas import tpu as pltpu
from jax.experimental.pallas import tpu_sc as plsc

_L = 16
_CH = 128


def _make_sc_kernel(B: int, C: int, NC: int, NS: int):
    NW = NC * NS
    per_w = B // NW
    n_ch = per_w // _CH
    n_vec = per_w // _L
    mesh = plsc.VectorSubcoreMesh(core_axis_name="c", subcore_axis_name="s")

    @functools.partial(
        pl.kernel,
        mesh=mesh,
        out_type=jax.ShapeDtypeStruct((NW, _L), jnp.float32),
        scratch_types=[
            pltpu.VMEM((per_w,), jnp.int32),
            pltpu.VMEM((per_w,), jnp.float32),
            pltpu.VMEM((n_ch, _CH), jnp.int32),
            pltpu.VMEM((per_w,), jnp.float32),
            pltpu.VMEM((_L,), jnp.float32),
            pltpu.SemaphoreType.DMA,
        ],
    )
    def sc_kernel(preds_hbm, tgt_hbm, mar_hbm, out_hbm,
                  tgt_v, mar_v, idx_v, gat_v, acc_v, sem):
        wid = lax.axis_index("s") * NC + lax.axis_index("c")
        base = wid * per_w
        pltpu.sync_copy(tgt_hbm.at[pl.ds(base, per_w)], tgt_v)
        pltpu.sync_copy(mar_hbm.at[pl.ds(base, per_w)], mar_v)

        lane = lax.iota(jnp.int32, _L)
        for step in range(n_vec):
            j = step // (_CH // _L)
            k = step % (_CH // _L)
            t = tgt_v[pl.ds(step * _L, _L)]
            rows = (base + step * _L) + lane
            idx_v[j, pl.ds(k * _L, _L)] = rows * C + t

        copies = [
            pltpu.async_copy(
                preds_hbm.at[idx_v.at[j]], gat_v.at[pl.ds(j * _CH, _CH)], sem
            )
            for j in range(n_ch)
        ]
        for c in copies:
            c.wait()

        acc = jnp.zeros((_L,), jnp.float32)
        for step in range(n_vec):
            g = gat_v[pl.ds(step * _L, _L)]
            m = mar_v[pl.ds(step * _L, _L)]
            w = jnp.exp(-0.5 * m * m)
            nz = (m > 0) | (m < 0)
            acc = acc + jnp.where(nz, w, 0.0) * g
        acc_v[...] = acc
        pltpu.sync_copy(acc_v, out_hbm.at[wid])

    return sc_kernel


def kernel(preds, targets, margin):
    B, C = preds.shape
    info = plsc.get_sparse_core_info()
    NC, NS = info.num_cores, info.num_subcores
    sc_kernel = _make_sc_kernel(B, C, NC, NS)
    partials = sc_kernel(
        preds.reshape(B * C),
        targets.astype(jnp.int32),
        margin,
    )
    return -jnp.sum(partials) / B


# speedup vs baseline: 1.5071x; 1.5071x over previous
import functools

import jax
import jax.numpy as jnp
from jax import lax
from jax.experimental import pallas as pl
from jax.experimental.pallas import tpu as pltpu
from jax.experimental.pallas import tpu_sc as plsc

_L = 16
_W = 128
_K = 112


def _make_sc_kernel(B: int, C: int, NC: int, NS: int):
    NW = NC * NS
    per_w = B // NW
    NB = (C + _W - 1) // _W
    n_vec = per_w // _L
    mesh = plsc.VectorSubcoreMesh(core_axis_name="c", subcore_axis_name="s")

    @functools.partial(
        pl.kernel,
        mesh=mesh,
        out_type=jax.ShapeDtypeStruct((NW, _L), jnp.float32),
        compiler_params=pltpu.CompilerParams(needs_layout_passes=False),
        scratch_types=[
            pltpu.VMEM((per_w,), jnp.int32),
            pltpu.VMEM((per_w,), jnp.float32),
            pltpu.VMEM((per_w,), jnp.int32),
            pltpu.VMEM((NB, _K), jnp.int32),
            pltpu.VMEM((NB * _K, _W), jnp.float32),
            pltpu.VMEM((_L,), jnp.float32),
            pltpu.SemaphoreType.DMA,
        ],
    )
    def sc_kernel(preds_hbm, tgt_hbm, mar_hbm, out_hbm,
                  tgt_v, mar_v, loc_v, idx_v, buf_v, acc_v, sem):
        wid = lax.axis_index("s") * NC + lax.axis_index("c")
        base = wid * per_w
        pltpu.sync_copy(tgt_hbm.at[pl.ds(base, per_w)], tgt_v)
        pltpu.sync_copy(mar_hbm.at[pl.ds(base, per_w)], mar_v)

        base_vec = jnp.full((_L,), 0, jnp.int32) + base
        for b in range(NB):
            def memset_body(c, carry, _b=b):
                idx_v[_b, pl.ds(c * _L, _L)] = base_vec
                return carry
            lax.fori_loop(0, _K // _L, memset_body, 0)

        lane = lax.iota(jnp.int32, _L)

        def compact_body(step, cnts):
            t = tgt_v[pl.ds(step * _L, _L)]
            blk = lax.shift_right_logical(t, 7)
            local = step * _L + lane
            rowc = jnp.zeros((_L,), jnp.int32)
            new_cnts = []
            for b in range(NB):
                mask = blk == b
                pref = lax.cumsum(mask.astype(jnp.int32), axis=0)
                pos = cnts[b] + pref - 1
                posc = jnp.minimum(pos, _K - 1)
                plsc.store_scatter(
                    idx_v, [jnp.full((_L,), b, jnp.int32), posc],
                    base_vec + local, mask=mask,
                )
                rowc = jnp.where(mask, b * _K + posc, rowc)
                new_cnts.append(cnts[b] + plsc.all_reduce_population_count(mask))
            loc_v[pl.ds(step * _L, _L)] = rowc
            return tuple(new_cnts)

        lax.fori_loop(
            0, n_vec, compact_body,
            tuple(jnp.zeros((_L,), jnp.int32) for _ in range(NB)),
        )

        zero_t = 0 * wid
        copies = []
        for b in range(NB):
            col0 = pl.multiple_of(b * _W + zero_t, _W)
            copies.append(pltpu.async_copy(
                preds_hbm.at[idx_v.at[b], pl.ds(col0, _W)],
                buf_v.at[pl.ds(b * _K, _K)],
                sem,
            ))
        for cp in copies:
            cp.wait()

        def extract_body(step, acc):
            rowc = loc_v[pl.ds(step * _L, _L)]
            t = tgt_v[pl.ds(step * _L, _L)]
            ln = lax.bitwise_and(t, _W - 1)
            g = plsc.load_gather(buf_v, [rowc, ln])
            m = mar_v[pl.ds(step * _L, _L)]
            w = jnp.exp(-0.5 * m * m)
            nz = (m > 0) | (m < 0)
            return acc + jnp.where(nz, w, 0.0) * g

        acc = lax.fori_loop(0, n_vec, extract_body, jnp.zeros((_L,), jnp.float32))
        acc_v[...] = acc
        pltpu.sync_copy(acc_v, out_hbm.at[wid])

    return sc_kernel


def kernel(preds, targets, margin):
    B, C = preds.shape
    info = plsc.get_sparse_core_info()
    NC, NS = info.num_cores, info.num_subcores
    sc_kernel = _make_sc_kernel(B, C, NC, NS)
    partials = sc_kernel(preds, targets.astype(jnp.int32), margin)
    return -jnp.sum(partials) / B


# speedup vs baseline: 4.8752x; 3.2347x over previous
import functools

import jax
import jax.numpy as jnp
from jax import lax
from jax.experimental import pallas as pl
from jax.experimental.pallas import tpu as pltpu
from jax.experimental.pallas import tpu_sc as plsc

_L = 16
_W = 128


def _make_sc_kernel(B: int, C: int, NC: int, NS: int):
    NW = NC * NS
    per_w = B // NW
    n_blk = per_w // _W
    n_vec = per_w // _L
    mesh = plsc.VectorSubcoreMesh(core_axis_name="c", subcore_axis_name="s")

    @functools.partial(
        pl.kernel,
        mesh=mesh,
        out_type=jax.ShapeDtypeStruct((NW, _L), jnp.float32),
        compiler_params=pltpu.CompilerParams(needs_layout_passes=False),
        scratch_types=[
            pltpu.VMEM((n_blk, _W), jnp.int32),
            pltpu.VMEM((per_w,), jnp.float32),
            pltpu.VMEM((per_w, _W), jnp.float32),
            pltpu.VMEM((_L,), jnp.float32),
            pltpu.SemaphoreType.DMA,
        ],
    )
    def sc_kernel(pt_hbm, tgt_hbm, mar_hbm, out_hbm,
                  tgt_v, mar_v, buf_v, acc_v, sem):
        wid = lax.axis_index("s") * NC + lax.axis_index("c")
        base = wid * per_w
        for sub in range(n_blk):
            pltpu.sync_copy(tgt_hbm.at[pl.ds(base + sub * _W, _W)], tgt_v.at[sub])
        pltpu.sync_copy(mar_hbm.at[pl.ds(base, per_w)], mar_v)

        copies = []
        for sub in range(n_blk):
            copies.append(pltpu.async_copy(
                pt_hbm.at[tgt_v.at[sub], pl.ds(base + sub * _W, _W)],
                buf_v.at[pl.ds(sub * _W, _W)],
                sem,
            ))
        for cp in copies:
            cp.wait()

        lane = lax.iota(jnp.int32, _L)

        def extract_body(step, acc):
            rowc = step * _L + lane
            ln = lax.bitwise_and(rowc, _W - 1)
            g = plsc.load_gather(buf_v, [rowc, ln])
            m = mar_v[pl.ds(step * _L, _L)]
            w = jnp.exp(-0.5 * m * m)
            nz = (m > 0) | (m < 0)
            return acc + jnp.where(nz, w, 0.0) * g

        acc = lax.fori_loop(0, n_vec, extract_body, jnp.zeros((_L,), jnp.float32))
        acc_v[...] = acc
        pltpu.sync_copy(acc_v, out_hbm.at[wid])

    return sc_kernel


def kernel(preds, targets, margin):
    B, C = preds.shape
    info = plsc.get_sparse_core_info()
    NC, NS = info.num_cores, info.num_subcores
    sc_kernel = _make_sc_kernel(B, C, NC, NS)
    partials = sc_kernel(preds.T, targets.astype(jnp.int32), margin)
    return -jnp.sum(partials) / B


# speedup vs baseline: 4.9333x; 1.0119x over previous
import functools

import jax
import jax.numpy as jnp
from jax import lax
from jax.experimental import pallas as pl
from jax.experimental.pallas import tpu as pltpu
from jax.experimental.pallas import tpu_sc as plsc

_L = 16
_W = 128


def _make_sc_kernel(B: int, C: int, NC: int, NS: int):
    NW = NC * NS
    per_w = B // NW
    n_blk = per_w // _W
    n_vec = per_w // _L
    mesh = plsc.VectorSubcoreMesh(core_axis_name="c", subcore_axis_name="s")

    @functools.partial(
        pl.kernel,
        mesh=mesh,
        out_type=jax.ShapeDtypeStruct((NW, _L), jnp.float32),
        compiler_params=pltpu.CompilerParams(needs_layout_passes=False),
        scratch_types=[
            pltpu.VMEM((n_blk, _W), jnp.int32),
            pltpu.VMEM((per_w,), jnp.float32),
            pltpu.VMEM((per_w, _W), jnp.float32),
            pltpu.VMEM((_L,), jnp.float32),
            pltpu.SemaphoreType.DMA,
            pltpu.SemaphoreType.DMA,
        ],
    )
    def sc_kernel(pt_hbm, tgt_hbm, mar_hbm, out_hbm,
                  tgt_v, mar_v, buf_v, acc_v, sem, msem):
        wid = lax.axis_index("s") * NC + lax.axis_index("c")
        base = wid * per_w
        mar_cp = pltpu.async_copy(mar_hbm.at[pl.ds(base, per_w)], mar_v, msem)
        for sub in range(n_blk):
            pltpu.sync_copy(tgt_hbm.at[pl.ds(base + sub * _W, _W)], tgt_v.at[sub])

        copies = []
        for sub in range(n_blk):
            copies.append(pltpu.async_copy(
                pt_hbm.at[tgt_v.at[sub], pl.ds(base + sub * _W, _W)],
                buf_v.at[pl.ds(sub * _W, _W)],
                sem,
            ))
        mar_cp.wait()

        lane = lax.iota(jnp.int32, _L)
        steps_per_blk = _W // _L

        def extract_body(step, acc):
            rowc = step * _L + lane
            ln = lax.bitwise_and(rowc, _W - 1)
            g = plsc.load_gather(buf_v, [rowc, ln])
            m = mar_v[pl.ds(step * _L, _L)]
            w = jnp.exp(-0.5 * m * m)
            nz = (m > 0) | (m < 0)
            return acc + jnp.where(nz, w, 0.0) * g

        acc = jnp.zeros((_L,), jnp.float32)
        for sub in range(n_blk):
            copies[sub].wait()
            acc = lax.fori_loop(
                sub * steps_per_blk, (sub + 1) * steps_per_blk, extract_body, acc
            )
        acc_v[...] = acc
        pltpu.sync_copy(acc_v, out_hbm.at[wid])

    return sc_kernel


def kernel(preds, targets, margin):
    B, C = preds.shape
    info = plsc.get_sparse_core_info()
    NC, NS = info.num_cores, info.num_subcores
    sc_kernel = _make_sc_kernel(B, C, NC, NS)
    partials = sc_kernel(preds.T, targets.astype(jnp.int32), margin)
    return -jnp.sum(partials) / B


# speedup vs baseline: 5.1922x; 1.0525x over previous
import functools

import jax
import jax.numpy as jnp
from jax import lax
from jax.experimental import pallas as pl
from jax.experimental.pallas import tpu as pltpu
from jax.experimental.pallas import tpu_sc as plsc

_L = 16
_W = 128


def _make_sc_kernel(B: int, C: int, NC: int, NS: int):
    NW = NC * NS
    per_w = B // NW
    n_blk = per_w // _W
    n_vec = per_w // _L
    mesh = plsc.VectorSubcoreMesh(core_axis_name="c", subcore_axis_name="s")

    @functools.partial(
        pl.kernel,
        mesh=mesh,
        out_type=jax.ShapeDtypeStruct((NW, _L), jnp.float32),
        compiler_params=pltpu.CompilerParams(needs_layout_passes=False),
        scratch_types=[
            pltpu.VMEM((n_blk, _W), jnp.int32),
            pltpu.VMEM((per_w,), jnp.float32),
            pltpu.VMEM((per_w, _W), jnp.float32),
            pltpu.VMEM((_L,), jnp.float32),
            pltpu.SemaphoreType.DMA,
            pltpu.SemaphoreType.DMA,
        ],
    )
    def sc_kernel(pt_hbm, tgt_hbm, mar_hbm, out_hbm,
                  tgt_v, mar_v, buf_v, acc_v, sem, msem):
        wid = lax.axis_index("s") * NC + lax.axis_index("c")
        base = wid * per_w
        mar_cp = pltpu.async_copy(mar_hbm.at[pl.ds(base, per_w)], mar_v, msem)
        tgt_cps = [
            pltpu.async_copy(
                tgt_hbm.at[pl.ds(base + sub * _W, _W)], tgt_v.at[sub], sem
            )
            for sub in range(n_blk)
        ]
        for cp in tgt_cps:
            cp.wait()

        copies = []
        for sub in range(n_blk):
            copies.append(pltpu.async_copy(
                pt_hbm.at[tgt_v.at[sub], pl.ds(base + sub * _W, _W)],
                buf_v.at[pl.ds(sub * _W, _W)],
                sem,
            ))
        mar_cp.wait()

        lane = lax.iota(jnp.int32, _L)
        steps_per_blk = _W // _L

        def extract_body(step, acc):
            rowc = step * _L + lane
            ln = lax.bitwise_and(rowc, _W - 1)
            g = plsc.load_gather(buf_v, [rowc, ln])
            m = mar_v[pl.ds(step * _L, _L)]
            w = jnp.exp(-0.5 * m * m)
            nz = (m > 0) | (m < 0)
            return acc + jnp.where(nz, w, 0.0) * g

        acc = jnp.zeros((_L,), jnp.float32)
        for sub in range(n_blk):
            copies[sub].wait()
            acc = lax.fori_loop(
                sub * steps_per_blk, (sub + 1) * steps_per_blk, extract_body, acc
            )
        acc_v[...] = acc
        pltpu.sync_copy(acc_v, out_hbm.at[wid])

    return sc_kernel


def kernel(preds, targets, margin):
    B, C = preds.shape
    info = plsc.get_sparse_core_info()
    NC, NS = info.num_cores, info.num_subcores
    sc_kernel = _make_sc_kernel(B, C, NC, NS)
    partials = sc_kernel(preds.T, targets.astype(jnp.int32), margin)
    return -jnp.sum(partials) / B
